# baseline (speedup 1.0000x reference)
import jax
import jax.numpy as jnp
from jax.experimental import pallas as pl
from jax.experimental.pallas import tpu as pltpu

_INPUT = 2048
_HIDDEN = 16384
_K = 32
_B = 32
_BLK = 1024
_NBLK = _HIDDEN // _BLK
_CW = 64
_NCH = _BLK // _CW
_NSLOT = 8


def _body(x_ref, we_ref, b_ref, wd_ref, o_ref,
          h_s, cv_s, cg_s, t_ref, it_ref, hw_ref):
    i = pl.program_id(0)

    @pl.when(i < _NBLK)
    def _encode():
        acc = jax.lax.dot_general(
            x_ref[...], we_ref[...],
            (((1,), (1,)), ((), ())),
            preferred_element_type=jnp.float32,
        )
        blk = jnp.maximum(acc + b_ref[:, pl.ds(i * _BLK, _BLK)], 0.0)
        h_s[:, pl.ds(i * _BLK, _BLK)] = blk

        hb3 = blk.reshape(_B, _NCH, _CW)
        iota_e = jax.lax.broadcasted_iota(jnp.int32, (_B, _NCH, _CW), 2)
        iota_c = jax.lax.broadcasted_iota(jnp.int32, (_B, _NCH), 1)
        gbase = (i * _NCH + iota_c) * _CW
        cvs = []
        cgs = []
        for _ in range(_NSLOT):
            cm = jnp.max(hb3, axis=2)
            im = jnp.min(jnp.where(hb3 == cm[:, :, None], iota_e, _CW),
                         axis=2)
            hb3 = jnp.where(iota_e == im[:, :, None], -1.0, hb3)
            cvs.append(cm)
            cgs.append(gbase + im)
        cv = jnp.stack(cvs, axis=1).reshape(_B, _NSLOT * _NCH)
        cg = jnp.stack(cgs, axis=1).reshape(_B, _NSLOT * _NCH)
        cv_s[pl.ds(i, 1), :, :] = cv[None]
        cg_s[pl.ds(i, 1), :, :] = cg[None]

    @pl.when(i == _NBLK)
    def _topk():
        def cbody(j, carry):
            Cv, Cg = carry
            m = jnp.max(Cv, axis=(0, 2), keepdims=True)
            gi = jnp.min(jnp.where(Cv == m, Cg, _HIDDEN),
                         axis=(0, 2), keepdims=True)
            Cv = jnp.where((Cv == m) & (Cg == gi), -1.0, Cv)
            t_ref[...] = m.reshape(_B, 1)
            it_ref[...] = gi.reshape(_B, 1)
            return (Cv, Cg)

        jax.lax.fori_loop(0, _K, cbody, (cv_s[...], cg_s[...]))

        h = h_s[...]
        iota = jax.lax.broadcasted_iota(jnp.int32, (_B, _HIDDEN), 1)
        keep = (h > t_ref[...]) | ((h == t_ref[...]) & (iota <= it_ref[...]))
        cnt = jnp.sum(keep.astype(jnp.int32), axis=1)
        bad = jnp.any(cnt != _K)

        @pl.when(bad)
        def _fallback():
            hw_ref[...] = h_s[...]

            def body(j, carry):
                hw = hw_ref[...]
                m = jnp.max(hw, axis=1, keepdims=True)
                im = jnp.min(jnp.where(hw == m, iota, _HIDDEN),
                             axis=1, keepdims=True)
                hw_ref[...] = jnp.where(iota == im, -1.0, hw)
                t_ref[...] = m
                it_ref[...] = im
                return carry

            jax.lax.fori_loop(0, _K, body, 0)

    @pl.when(i >= _NBLK)
    def _decode():
        j = i - _NBLK
        hblk = h_s[:, pl.ds(j * _BLK, _BLK)]
        iota = jax.lax.broadcasted_iota(jnp.int32, (_B, _BLK), 1) + j * _BLK
        keep = (hblk > t_ref[...]) | ((hblk == t_ref[...]) &
                                      (iota <= it_ref[...]))
        hs = jnp.where(keep, hblk, 0.0)
        acc = jax.lax.dot_general(
            hs, wd_ref[...],
            (((1,), (1,)), ((), ())),
            preferred_element_type=jnp.float32,
        )

        @pl.when(i == _NBLK)
        def _init():
            o_ref[...] = acc

        @pl.when(i > _NBLK)
        def _acc():
            o_ref[...] += acc


def kernel(x, W_enc, b_enc, W_dec):
    b2 = b_enc.reshape(1, _HIDDEN)

    recon = pl.pallas_call(
        _body,
        grid=(2 * _NBLK,),
        in_specs=[
            pl.BlockSpec((_B, _INPUT), lambda i: (0, 0)),
            pl.BlockSpec((_BLK, _INPUT),
                         lambda i: (jnp.minimum(i, _NBLK - 1), 0)),
            pl.BlockSpec((1, _HIDDEN), lambda i: (0, 0)),
            pl.BlockSpec((_INPUT, _BLK),
                         lambda i: (0, jnp.maximum(i - _NBLK, 0))),
        ],
        out_specs=pl.BlockSpec((_B, _INPUT), lambda i: (0, 0)),
        out_shape=jax.ShapeDtypeStruct((_B, _INPUT), jnp.float32),
        scratch_shapes=[
            pltpu.VMEM((_B, _HIDDEN), jnp.float32),
            pltpu.VMEM((_NBLK, _B, _NSLOT * _NCH), jnp.float32),
            pltpu.VMEM((_NBLK, _B, _NSLOT * _NCH), jnp.int32),
            pltpu.VMEM((_B, 1), jnp.float32),
            pltpu.VMEM((_B, 1), jnp.int32),
            pltpu.VMEM((_B, _HIDDEN), jnp.float32),
        ],
    )(x, W_enc, b2, W_dec)

    return recon
